# Initial kernel scaffold; baseline (speedup 1.0000x reference)
import jax
import jax.numpy as jnp
from jax.experimental import pallas as pl


def kernel(input_ids, embed_table):
    raise NotImplementedError("write your pallas kernel here")


# speedup vs baseline: 1.4574x; 1.4574x over previous
import functools

import jax
import jax.numpy as jnp
from jax import lax
from jax.experimental import pallas as pl
from jax.experimental.pallas import tpu as pltpu
from jax.experimental.pallas import tpu_sc as plsc


@functools.cache
def _build(batch, seq_len, vocab, dim, idx_dtype):
    info = plsc.get_sparse_core_info()
    nc, ns, nl = info.num_cores, info.num_subcores, info.num_lanes
    nw = nc * ns
    n = batch * seq_len
    assert n % nw == 0
    per_w = n // nw
    chunk = 128
    assert per_w % chunk == 0
    n_chunk = per_w // chunk
    half_rows = per_w // 2
    chunks_per_half = n_chunk // 2

    mesh = plsc.VectorSubcoreMesh(core_axis_name="c", subcore_axis_name="s")

    @functools.partial(
        pl.kernel,
        mesh=mesh,
        out_type=(
            jax.ShapeDtypeStruct((n, dim), jnp.float32),
            jax.ShapeDtypeStruct((nw, per_w), idx_dtype),
        ),
        scratch_types=[
            pltpu.VMEM((n_chunk, chunk), jnp.int32),
            pltpu.VMEM((half_rows, dim), jnp.float32),
            pltpu.VMEM((per_w,), idx_dtype),
            pltpu.SemaphoreType.DMA,
        ],
    )
    def k(ids_hbm, table_hbm, out_hbm, pos_hbm, idx_v, rows_v, pos_v, sem):
        wid = lax.axis_index("s") * nc + lax.axis_index("c")
        base = wid * per_w
        pltpu.sync_copy(ids_hbm.at[wid], idx_v)
        for half in range(2):
            copies = [
                pltpu.async_copy(
                    table_hbm.at[idx_v.at[half * chunks_per_half + j]],
                    rows_v.at[pl.ds(j * chunk, chunk)],
                    sem,
                )
                for j in range(chunks_per_half)
            ]
            for cp in copies:
                cp.wait()
            pltpu.sync_copy(
                rows_v, out_hbm.at[pl.ds(base + half * half_rows, half_rows)]
            )
        row_base = lax.rem(base, seq_len)
        iota = lax.iota(idx_dtype, nl)
        for j in range(per_w // nl):
            pos_v[pl.ds(j * nl, nl)] = iota + (row_base + j * nl)
        pltpu.sync_copy(pos_v, pos_hbm.at[wid])

    return k, nw, n_chunk, chunk


def kernel(input_ids, embed_table):
    batch, seq_len = input_ids.shape
    vocab, dim = embed_table.shape
    k, nw, n_chunk, chunk = _build(batch, seq_len, vocab, dim, input_ids.dtype)
    ids3 = input_ids.reshape(nw, n_chunk, chunk)
    hidden, pos = k(ids3, embed_table)
    return hidden.reshape(batch, seq_len, dim), pos.reshape(batch, seq_len)
